# Initial kernel scaffold; baseline (speedup 1.0000x reference)
import jax
import jax.numpy as jnp
from jax.experimental import pallas as pl


def kernel(x, edge_index, edge_weights, W1, b1, g1, be1, W2, b2, g2, be2, W3, b3, Wl, bl):
    raise NotImplementedError("write your pallas kernel here")


# speedup vs baseline: 5.6984x; 5.6984x over previous
import jax
import jax.numpy as jnp
from jax import lax
from jax.experimental import pallas as pl
from jax.experimental.pallas import tpu as pltpu
from jax.experimental.pallas import tpu_sc as plsc

N = 10000
E = 320000
DIN = 128
HID = 256
NCLS = 64

NC = 2
NS = 16
LN = 16
NQ = 4
Q = HID // NQ

KD = 80
CHD = E // (NC * NS) // KD
KE = 80
CHE = E // NS // KE
NP = 10240
RZ = NP // NS
ZR = 128

_SC_PARAMS = pltpu.CompilerParams(
    needs_layout_passes=False, use_tc_tiling_on_sc=False)


def _bcast_ld(ref, row_idx, col):
  return plsc.load_gather(ref, [row_idx, jnp.full((LN,), col, jnp.int32)])


def _zero_rows(zbuf, nrows, ncols):
  zv = jnp.zeros((LN,), jnp.float32)

  def body(i, _):
    for f in range(ncols // LN):
      zbuf[i, pl.ds(f * LN, LN)] = zv
    return 0

  lax.fori_loop(0, nrows, body, 0)


RLAST = N - (NS - 1) * RZ


def _writeout(acc, out_ref, sid):
  @pl.when(sid < NS - 1)
  def _():
    pltpu.sync_copy(acc.at[pl.ds(sid * RZ, RZ)], out_ref.at[pl.ds(sid * RZ, RZ)])

  @pl.when(sid == NS - 1)
  def _():
    pltpu.sync_copy(acc.at[pl.ds((NS - 1) * RZ, RLAST)],
                    out_ref.at[pl.ds((NS - 1) * RZ, RLAST)])


def _sc_deg_body(dst_hbm, ew_hbm, out_hbm, dst_v, ew_v, val_v, z_v, acc):
  cid = lax.axis_index("c")
  sid = lax.axis_index("s")
  wid = sid * NC + cid

  _zero_rows(z_v, RZ, LN)
  pltpu.sync_copy(z_v, acc.at[pl.ds(sid * RZ, RZ)])
  pltpu.sync_copy(dst_hbm.at[wid], dst_v)
  pltpu.sync_copy(ew_hbm.at[wid], ew_v)
  plsc.subcore_barrier()

  def chunk(g, _):
    gv = jnp.full((LN,), g, jnp.int32)
    for e in range(KD):
      val_v[e] = _bcast_ld(ew_v, gv, e)
    pltpu.sync_copy(val_v, acc.at[dst_v.at[g]], add=True)
    return 0

  lax.fori_loop(0, CHD, chunk, 0)
  plsc.subcore_barrier()
  _writeout(acc, out_hbm.at[cid], sid)


_sc_deg = pl.kernel(
    _sc_deg_body,
    out_type=jax.ShapeDtypeStruct((NC, N, LN), jnp.float32),
    mesh=plsc.VectorSubcoreMesh(core_axis_name="c", subcore_axis_name="s"),
    compiler_params=_SC_PARAMS,
    scratch_types=[
        pltpu.VMEM((CHD, KD), jnp.int32),
        pltpu.VMEM((CHD, KD), jnp.float32),
        pltpu.VMEM((KD, LN), jnp.float32),
        pltpu.VMEM((RZ, LN), jnp.float32),
        pltpu.VMEM_SHARED((NP, LN), jnp.float32),
    ],
)


def _sc_spmm_body(hq0, hq1, hq2, hq3, src_hbm, dst_hbm, ew_hbm, out_hbm,
                  src_v, dst_v, ew_v, rows_v, z_v, acc):
  cid = lax.axis_index("c")
  sid = lax.axis_index("s")

  pltpu.sync_copy(src_hbm.at[sid], src_v)
  pltpu.sync_copy(dst_hbm.at[sid], dst_v)
  pltpu.sync_copy(ew_hbm.at[sid], ew_v)
  _zero_rows(z_v, ZR, Q)

  def run(hs_ref, qi):
    for r in range(RZ // ZR):
      pltpu.sync_copy(z_v, acc.at[pl.ds(sid * RZ + r * ZR, ZR)])
    plsc.subcore_barrier()

    def chunk(g, _):
      pltpu.sync_copy(hs_ref.at[src_v.at[g]], rows_v)
      gv = jnp.full((LN,), g, jnp.int32)
      for e in range(KE):
        w = _bcast_ld(ew_v, gv, e)
        for f in range(Q // LN):
          rows_v[e, pl.ds(f * LN, LN)] = rows_v[e, pl.ds(f * LN, LN)] * w
      pltpu.sync_copy(rows_v, acc.at[dst_v.at[g]], add=True)
      return 0

    lax.fori_loop(0, CHE, chunk, 0)
    plsc.subcore_barrier()
    _writeout(acc, out_hbm.at[qi], sid)

  quarters = (hq0, hq1, hq2, hq3)
  for p in range(2):
    for c in range(NC):
      @pl.when(cid == c)
      def _(p=p, c=c):
        run(quarters[2 * p + c], 2 * p + c)


_sc_spmm = pl.kernel(
    _sc_spmm_body,
    out_type=jax.ShapeDtypeStruct((NQ, N, Q), jnp.float32),
    mesh=plsc.VectorSubcoreMesh(core_axis_name="c", subcore_axis_name="s"),
    compiler_params=_SC_PARAMS,
    scratch_types=[
        pltpu.VMEM((CHE, KE), jnp.int32),
        pltpu.VMEM((CHE, KE), jnp.int32),
        pltpu.VMEM((CHE, KE), jnp.float32),
        pltpu.VMEM((KE, Q), jnp.float32),
        pltpu.VMEM((ZR, Q), jnp.float32),
        pltpu.VMEM_SHARED((NP, Q), jnp.float32),
    ],
)


BR = 2000
NB = N // BR


def _lrelu(h):
  return jnp.where(h >= 0, h, 0.01 * h)


def _row_spec(cols):
  return pl.BlockSpec((BR, cols), lambda i: (i, 0))


def _full_spec(shape):
  nd = len(shape)
  return pl.BlockSpec(shape, lambda i: (0,) * nd)


_QSPECS = [_row_spec(Q) for _ in range(NQ)]


def _split_q(hs, hs_refs):
  for q in range(NQ):
    hs_refs[q][...] = hs[:, q * Q:(q + 1) * Q]


def _tc1_body(degp_ref, x_ref, w1_ref, dis_ref, h0, h1, h2, h3):
  degs = degp_ref[0] + degp_ref[1]
  deg = 1.0 + degs[:, 0:1]
  dis = jnp.where(deg > 0, lax.rsqrt(deg), 0.0)
  dis_ref[...] = dis
  xw = jnp.dot(x_ref[...], w1_ref[...], preferred_element_type=jnp.float32)
  _split_q(xw * dis, (h0, h1, h2, h3))


_tc1 = pl.pallas_call(
    _tc1_body,
    grid=(NB,),
    in_specs=[
        pl.BlockSpec((NC, BR, LN), lambda i: (0, i, 0)),
        _row_spec(DIN),
        _full_spec((DIN, HID)),
    ],
    out_specs=[_row_spec(1)] + _QSPECS,
    out_shape=(
        jax.ShapeDtypeStruct((N, 1), jnp.float32),
    ) + tuple(jax.ShapeDtypeStruct((N, Q), jnp.float32) for _ in range(NQ)),
)


def _tc_bn1_body(s_ref, hq0, hq1, hq2, hq3, dis_ref, b_ref, a_ref, sums_ref):
  i = pl.program_id(0)
  sc = jnp.concatenate([s_ref[q] for q in range(NQ)], axis=1)
  hc = jnp.concatenate([h[...] for h in (hq0, hq1, hq2, hq3)], axis=1)
  pre = dis_ref[...] * (sc + hc) + b_ref[...][None, :]
  a = _lrelu(pre)
  a_ref[...] = a

  @pl.when(i == 0)
  def _():
    sums_ref[...] = jnp.zeros((8, HID), jnp.float32)

  sums_ref[0:1, :] = sums_ref[0:1, :] + jnp.sum(a, axis=0, keepdims=True)
  sums_ref[1:2, :] = sums_ref[1:2, :] + jnp.sum(a * a, axis=0, keepdims=True)


_tc_bn1 = pl.pallas_call(
    _tc_bn1_body,
    grid=(NB,),
    in_specs=[
        pl.BlockSpec((NQ, BR, Q), lambda i: (0, i, 0)),
        *_QSPECS,
        _row_spec(1),
        _full_spec((HID,)),
    ],
    out_specs=[_row_spec(HID), _full_spec((8, HID))],
    out_shape=(
        jax.ShapeDtypeStruct((N, HID), jnp.float32),
        jax.ShapeDtypeStruct((8, HID), jnp.float32),
    ),
)


def _tc_bn2_body(a_ref, sums_ref, dis_ref, g_ref, be_ref, w_ref,
                 out_ref, n0, n1, n2, n3):
  mu = sums_ref[0:1, :] / N
  var = sums_ref[1:2, :] / N - mu * mu
  o = ((a_ref[...] - mu) * lax.rsqrt(var + 1e-5) * g_ref[...][None, :]
       + be_ref[...][None, :])
  out_ref[...] = o
  hsn = jnp.dot(o, w_ref[...], preferred_element_type=jnp.float32)
  _split_q(hsn * dis_ref[...], (n0, n1, n2, n3))


_tc_bn2 = pl.pallas_call(
    _tc_bn2_body,
    grid=(NB,),
    in_specs=[
        _row_spec(HID),
        _full_spec((8, HID)),
        _row_spec(1),
        _full_spec((HID,)),
        _full_spec((HID,)),
        _full_spec((HID, HID)),
    ],
    out_specs=[_row_spec(HID)] + _QSPECS,
    out_shape=(
        jax.ShapeDtypeStruct((N, HID), jnp.float32),
    ) + tuple(jax.ShapeDtypeStruct((N, Q), jnp.float32) for _ in range(NQ)),
)


def _tc_fin_body(s_ref, hq0, hq1, hq2, hq3, dis_ref, b_ref, o1_ref, o2_ref,
                 wl_ref, bl_ref, res_ref):
  sc = jnp.concatenate([s_ref[q] for q in range(NQ)], axis=1)
  hc = jnp.concatenate([h[...] for h in (hq0, hq1, hq2, hq3)], axis=1)
  o3 = _lrelu(dis_ref[...] * (sc + hc) + b_ref[...][None, :])
  wl = wl_ref[...]
  res = (jnp.dot(o1_ref[...], wl[:HID], preferred_element_type=jnp.float32)
         + jnp.dot(o2_ref[...], wl[HID:2 * HID],
                   preferred_element_type=jnp.float32)
         + jnp.dot(o3, wl[2 * HID:], preferred_element_type=jnp.float32)
         + bl_ref[...][None, :])
  res_ref[...] = res


_tc_fin = pl.pallas_call(
    _tc_fin_body,
    grid=(NB,),
    in_specs=[
        pl.BlockSpec((NQ, BR, Q), lambda i: (0, i, 0)),
        *_QSPECS,
        _row_spec(1),
        _full_spec((HID,)),
        _row_spec(HID),
        _row_spec(HID),
        _full_spec((3 * HID, NCLS)),
        _full_spec((NCLS,)),
    ],
    out_specs=_row_spec(NCLS),
    out_shape=jax.ShapeDtypeStruct((N, NCLS), jnp.float32),
)


def kernel(x, edge_index, edge_weights, W1, b1, g1, be1, W2, b2, g2, be2,
           W3, b3, Wl, bl):
  ei = edge_index.astype(jnp.int32)
  src = ei[0]
  dst = ei[1]
  ew = edge_weights.astype(jnp.float32)

  dst_d = dst.reshape(NC * NS, CHD, KD)
  ew_d = ew.reshape(NC * NS, CHD, KD)
  src_s = src.reshape(NS, CHE, KE)
  dst_s = dst.reshape(NS, CHE, KE)
  ew_s = ew.reshape(NS, CHE, KE)

  degp = _sc_deg(dst_d, ew_d)
  dis, *hs1 = _tc1(degp, x, W1)

  s1 = _sc_spmm(*hs1, src_s, dst_s, ew_s)
  a1, sums1 = _tc_bn1(s1, *hs1, dis, b1)
  out1, *hs2 = _tc_bn2(a1, sums1, dis, g1, be1, W2)

  s2 = _sc_spmm(*hs2, src_s, dst_s, ew_s)
  a2, sums2 = _tc_bn1(s2, *hs2, dis, b2)
  out2, *hs3 = _tc_bn2(a2, sums2, dis, g2, be2, W3)

  s3 = _sc_spmm(*hs3, src_s, dst_s, ew_s)
  return _tc_fin(s3, *hs3, dis, b3, out1, out2, Wl, bl)


# speedup vs baseline: 9.8476x; 1.7281x over previous
import jax
import jax.numpy as jnp
from jax import lax
from jax.experimental import pallas as pl
from jax.experimental.pallas import tpu as pltpu
from jax.experimental.pallas import tpu_sc as plsc

N = 10000
E = 320000
DIN = 128
HID = 256
NCLS = 64

NC = 2
NS = 16
LN = 16
NQ = 4
Q = HID // NQ

KD = 80
CHD = E // (NC * NS) // KD
KE = 100
CHE = E // NS // KE
NP = 10240
RZ = NP // NS
ZR = 128

_SC_PARAMS = pltpu.CompilerParams(
    needs_layout_passes=False, use_tc_tiling_on_sc=False)


def _bcast_ld(ref, row_idx, col):
  return plsc.load_gather(ref, [row_idx, jnp.full((LN,), col, jnp.int32)])


def _zero_rows(zbuf, nrows, ncols):
  zv = jnp.zeros((LN,), jnp.float32)

  def body(i, _):
    for f in range(ncols // LN):
      zbuf[i, pl.ds(f * LN, LN)] = zv
    return 0

  lax.fori_loop(0, nrows, body, 0)


RLAST = N - (NS - 1) * RZ


def _writeout(acc, out_ref, sid):
  @pl.when(sid < NS - 1)
  def _():
    pltpu.sync_copy(acc.at[pl.ds(sid * RZ, RZ)], out_ref.at[pl.ds(sid * RZ, RZ)])

  @pl.when(sid == NS - 1)
  def _():
    pltpu.sync_copy(acc.at[pl.ds((NS - 1) * RZ, RLAST)],
                    out_ref.at[pl.ds((NS - 1) * RZ, RLAST)])


def _sc_deg_body(dst_hbm, ew_hbm, out_hbm, dst_v, ew_v, val_v, z_v, acc):
  cid = lax.axis_index("c")
  sid = lax.axis_index("s")
  wid = sid * NC + cid

  _zero_rows(z_v, RZ, LN)
  pltpu.sync_copy(z_v, acc.at[pl.ds(sid * RZ, RZ)])
  pltpu.sync_copy(dst_hbm.at[wid], dst_v)
  pltpu.sync_copy(ew_hbm.at[wid], ew_v)
  plsc.subcore_barrier()

  def chunk(g, _):
    gv = jnp.full((LN,), g, jnp.int32)
    for e in range(KD):
      val_v[e] = _bcast_ld(ew_v, gv, e)
    pltpu.sync_copy(val_v, acc.at[dst_v.at[g]], add=True)
    return 0

  lax.fori_loop(0, CHD, chunk, 0)
  plsc.subcore_barrier()
  _writeout(acc, out_hbm.at[cid], sid)


_sc_deg = pl.kernel(
    _sc_deg_body,
    out_type=jax.ShapeDtypeStruct((NC, N, LN), jnp.float32),
    mesh=plsc.VectorSubcoreMesh(core_axis_name="c", subcore_axis_name="s"),
    compiler_params=_SC_PARAMS,
    scratch_types=[
        pltpu.VMEM((CHD, KD), jnp.int32),
        pltpu.VMEM((CHD, KD), jnp.float32),
        pltpu.VMEM((KD, LN), jnp.float32),
        pltpu.VMEM((RZ, LN), jnp.float32),
        pltpu.VMEM_SHARED((NP, LN), jnp.float32),
    ],
)


def _sc_spmm_body(hq0, hq1, hq2, hq3, src_hbm, dst_hbm, ew_hbm, out_hbm,
                  src_v, dst_v, ew_v, rows_v0, rows_v1, sem0, sem1, z_v, acc):
  cid = lax.axis_index("c")
  sid = lax.axis_index("s")

  pltpu.sync_copy(src_hbm.at[sid], src_v)
  pltpu.sync_copy(dst_hbm.at[sid], dst_v)
  pltpu.sync_copy(ew_hbm.at[sid], ew_v)
  _zero_rows(z_v, ZR, Q)
  bufs = ((rows_v0, sem0), (rows_v1, sem1))

  def run(hs_ref, qi):
    for r in range(RZ // ZR):
      pltpu.sync_copy(z_v, acc.at[pl.ds(sid * RZ + r * ZR, ZR)])
    plsc.subcore_barrier()

    pltpu.async_copy(hs_ref.at[src_v.at[0]], rows_v0, sem0)

    def pair(h, _):
      for b, (rows_b, sem_b) in enumerate(bufs):
        g = h * 2 + b
        nrows, nsem = bufs[1 - b]

        @pl.when(g + 1 < CHE)
        def _():
          pltpu.async_copy(hs_ref.at[src_v.at[g + 1]], nrows, nsem)

        pltpu.make_async_copy(hs_ref.at[src_v.at[g]], rows_b, sem_b).wait()
        gv = jnp.full((LN,), g, jnp.int32)

        def sbody(it, _, rows_b=rows_b):
          for u in range(4):
            e = it * 4 + u
            w = _bcast_ld(ew_v, gv, e)
            for f in range(Q // LN):
              rows_b[e, pl.ds(f * LN, LN)] = rows_b[e, pl.ds(f * LN, LN)] * w
          return 0

        lax.fori_loop(0, KE // 4, sbody, 0)
        pltpu.sync_copy(rows_b, acc.at[dst_v.at[g]], add=True)
      return 0

    lax.fori_loop(0, CHE // 2, pair, 0)
    plsc.subcore_barrier()
    _writeout(acc, out_hbm.at[qi], sid)

  quarters = (hq0, hq1, hq2, hq3)
  for p in range(2):
    for c in range(NC):
      @pl.when(cid == c)
      def _(p=p, c=c):
        run(quarters[2 * p + c], 2 * p + c)


_sc_spmm = pl.kernel(
    _sc_spmm_body,
    out_type=jax.ShapeDtypeStruct((NQ, N, Q), jnp.float32),
    mesh=plsc.VectorSubcoreMesh(core_axis_name="c", subcore_axis_name="s"),
    compiler_params=_SC_PARAMS,
    scratch_types=[
        pltpu.VMEM((CHE, KE), jnp.int32),
        pltpu.VMEM((CHE, KE), jnp.int32),
        pltpu.VMEM((CHE, KE), jnp.float32),
        pltpu.VMEM((KE, Q), jnp.float32),
        pltpu.VMEM((KE, Q), jnp.float32),
        pltpu.SemaphoreType.DMA,
        pltpu.SemaphoreType.DMA,
        pltpu.VMEM((ZR, Q), jnp.float32),
        pltpu.VMEM_SHARED((NP, Q), jnp.float32),
    ],
)


BR = 2000
NB = N // BR


def _lrelu(h):
  return jnp.where(h >= 0, h, 0.01 * h)


def _row_spec(cols):
  return pl.BlockSpec((BR, cols), lambda i: (i, 0))


def _full_spec(shape):
  nd = len(shape)
  return pl.BlockSpec(shape, lambda i: (0,) * nd)


_QSPECS = [_row_spec(Q) for _ in range(NQ)]


def _split_q(hs, hs_refs):
  for q in range(NQ):
    hs_refs[q][...] = hs[:, q * Q:(q + 1) * Q]


def _tc1_body(degp_ref, x_ref, w1_ref, dis_ref, h0, h1, h2, h3):
  degs = degp_ref[0] + degp_ref[1]
  deg = 1.0 + degs[:, 0:1]
  dis = jnp.where(deg > 0, lax.rsqrt(deg), 0.0)
  dis_ref[...] = dis
  xw = jnp.dot(x_ref[...], w1_ref[...], preferred_element_type=jnp.float32)
  _split_q(xw * dis, (h0, h1, h2, h3))


_tc1 = pl.pallas_call(
    _tc1_body,
    grid=(NB,),
    in_specs=[
        pl.BlockSpec((NC, BR, LN), lambda i: (0, i, 0)),
        _row_spec(DIN),
        _full_spec((DIN, HID)),
    ],
    out_specs=[_row_spec(1)] + _QSPECS,
    out_shape=(
        jax.ShapeDtypeStruct((N, 1), jnp.float32),
    ) + tuple(jax.ShapeDtypeStruct((N, Q), jnp.float32) for _ in range(NQ)),
)


def _tc_bn1_body(s_ref, hq0, hq1, hq2, hq3, dis_ref, b_ref, a_ref, sums_ref):
  i = pl.program_id(0)
  sc = jnp.concatenate([s_ref[q] for q in range(NQ)], axis=1)
  hc = jnp.concatenate([h[...] for h in (hq0, hq1, hq2, hq3)], axis=1)
  pre = dis_ref[...] * (sc + hc) + b_ref[...][None, :]
  a = _lrelu(pre)
  a_ref[...] = a

  @pl.when(i == 0)
  def _():
    sums_ref[...] = jnp.zeros((8, HID), jnp.float32)

  sums_ref[0:1, :] = sums_ref[0:1, :] + jnp.sum(a, axis=0, keepdims=True)
  sums_ref[1:2, :] = sums_ref[1:2, :] + jnp.sum(a * a, axis=0, keepdims=True)


_tc_bn1 = pl.pallas_call(
    _tc_bn1_body,
    grid=(NB,),
    in_specs=[
        pl.BlockSpec((NQ, BR, Q), lambda i: (0, i, 0)),
        *_QSPECS,
        _row_spec(1),
        _full_spec((HID,)),
    ],
    out_specs=[_row_spec(HID), _full_spec((8, HID))],
    out_shape=(
        jax.ShapeDtypeStruct((N, HID), jnp.float32),
        jax.ShapeDtypeStruct((8, HID), jnp.float32),
    ),
)


def _tc_bn2_body(a_ref, sums_ref, dis_ref, g_ref, be_ref, w_ref,
                 out_ref, n0, n1, n2, n3):
  mu = sums_ref[0:1, :] / N
  var = sums_ref[1:2, :] / N - mu * mu
  o = ((a_ref[...] - mu) * lax.rsqrt(var + 1e-5) * g_ref[...][None, :]
       + be_ref[...][None, :])
  out_ref[...] = o
  hsn = jnp.dot(o, w_ref[...], preferred_element_type=jnp.float32)
  _split_q(hsn * dis_ref[...], (n0, n1, n2, n3))


_tc_bn2 = pl.pallas_call(
    _tc_bn2_body,
    grid=(NB,),
    in_specs=[
        _row_spec(HID),
        _full_spec((8, HID)),
        _row_spec(1),
        _full_spec((HID,)),
        _full_spec((HID,)),
        _full_spec((HID, HID)),
    ],
    out_specs=[_row_spec(HID)] + _QSPECS,
    out_shape=(
        jax.ShapeDtypeStruct((N, HID), jnp.float32),
    ) + tuple(jax.ShapeDtypeStruct((N, Q), jnp.float32) for _ in range(NQ)),
)


def _tc_fin_body(s_ref, hq0, hq1, hq2, hq3, dis_ref, b_ref, o1_ref, o2_ref,
                 wl_ref, bl_ref, res_ref):
  sc = jnp.concatenate([s_ref[q] for q in range(NQ)], axis=1)
  hc = jnp.concatenate([h[...] for h in (hq0, hq1, hq2, hq3)], axis=1)
  o3 = _lrelu(dis_ref[...] * (sc + hc) + b_ref[...][None, :])
  wl = wl_ref[...]
  res = (jnp.dot(o1_ref[...], wl[:HID], preferred_element_type=jnp.float32)
         + jnp.dot(o2_ref[...], wl[HID:2 * HID],
                   preferred_element_type=jnp.float32)
         + jnp.dot(o3, wl[2 * HID:], preferred_element_type=jnp.float32)
         + bl_ref[...][None, :])
  res_ref[...] = res


_tc_fin = pl.pallas_call(
    _tc_fin_body,
    grid=(NB,),
    in_specs=[
        pl.BlockSpec((NQ, BR, Q), lambda i: (0, i, 0)),
        *_QSPECS,
        _row_spec(1),
        _full_spec((HID,)),
        _row_spec(HID),
        _row_spec(HID),
        _full_spec((3 * HID, NCLS)),
        _full_spec((NCLS,)),
    ],
    out_specs=_row_spec(NCLS),
    out_shape=jax.ShapeDtypeStruct((N, NCLS), jnp.float32),
)


def kernel(x, edge_index, edge_weights, W1, b1, g1, be1, W2, b2, g2, be2,
           W3, b3, Wl, bl):
  ei = edge_index.astype(jnp.int32)
  src = ei[0]
  dst = ei[1]
  ew = edge_weights.astype(jnp.float32)

  dst_d = dst.reshape(NC * NS, CHD, KD)
  ew_d = ew.reshape(NC * NS, CHD, KD)
  src_s = src.reshape(NS, CHE, KE)
  dst_s = dst.reshape(NS, CHE, KE)
  ew_s = ew.reshape(NS, CHE, KE)

  degp = _sc_deg(dst_d, ew_d)
  dis, *hs1 = _tc1(degp, x, W1)

  s1 = _sc_spmm(*hs1, src_s, dst_s, ew_s)
  a1, sums1 = _tc_bn1(s1, *hs1, dis, b1)
  out1, *hs2 = _tc_bn2(a1, sums1, dis, g1, be1, W2)

  s2 = _sc_spmm(*hs2, src_s, dst_s, ew_s)
  a2, sums2 = _tc_bn1(s2, *hs2, dis, b2)
  out2, *hs3 = _tc_bn2(a2, sums2, dis, g2, be2, W3)

  s3 = _sc_spmm(*hs3, src_s, dst_s, ew_s)
  return _tc_fin(s3, *hs3, dis, b3, out1, out2, Wl, bl)


# speedup vs baseline: 9.9033x; 1.0056x over previous
import jax
import jax.numpy as jnp
from jax import lax
from jax.experimental import pallas as pl
from jax.experimental.pallas import tpu as pltpu
from jax.experimental.pallas import tpu_sc as plsc

N = 10000
E = 320000
DIN = 128
HID = 256
NCLS = 64

NC = 2
NS = 16
LN = 16
NQ = 4
Q = HID // NQ

KD = 80
CHD = E // (NC * NS) // KD
KE = 100
CHE = E // NS // KE
NP = 10240
RZ = NP // NS
ZR = 128

_SC_PARAMS = pltpu.CompilerParams(
    needs_layout_passes=False, use_tc_tiling_on_sc=False)


def _bcast_ld(ref, row_idx, col):
  return plsc.load_gather(ref, [row_idx, jnp.full((LN,), col, jnp.int32)])


def _zero_rows(zbuf, nrows, ncols):
  zv = jnp.zeros((LN,), jnp.float32)

  def body(i, _):
    for f in range(ncols // LN):
      zbuf[i, pl.ds(f * LN, LN)] = zv
    return 0

  lax.fori_loop(0, nrows, body, 0)


RLAST = N - (NS - 1) * RZ


def _writeout(acc, out_ref, sid):
  @pl.when(sid < NS - 1)
  def _():
    pltpu.sync_copy(acc.at[pl.ds(sid * RZ, RZ)], out_ref.at[pl.ds(sid * RZ, RZ)])

  @pl.when(sid == NS - 1)
  def _():
    pltpu.sync_copy(acc.at[pl.ds((NS - 1) * RZ, RLAST)],
                    out_ref.at[pl.ds((NS - 1) * RZ, RLAST)])


def _sc_deg_body(dst_hbm, ew_hbm, out_hbm, dst_v, ew_v, val_v, z_v, acc):
  cid = lax.axis_index("c")
  sid = lax.axis_index("s")
  wid = sid * NC + cid

  _zero_rows(z_v, RZ, LN)
  pltpu.sync_copy(z_v, acc.at[pl.ds(sid * RZ, RZ)])
  pltpu.sync_copy(dst_hbm.at[wid], dst_v)
  pltpu.sync_copy(ew_hbm.at[wid], ew_v)
  plsc.subcore_barrier()

  def chunk(g, _):
    gv = jnp.full((LN,), g, jnp.int32)
    for e in range(KD):
      val_v[e] = _bcast_ld(ew_v, gv, e)
    pltpu.sync_copy(val_v, acc.at[dst_v.at[g]], add=True)
    return 0

  lax.fori_loop(0, CHD, chunk, 0)
  plsc.subcore_barrier()
  _writeout(acc, out_hbm.at[cid], sid)


_sc_deg = pl.kernel(
    _sc_deg_body,
    out_type=jax.ShapeDtypeStruct((NC, N, LN), jnp.float32),
    mesh=plsc.VectorSubcoreMesh(core_axis_name="c", subcore_axis_name="s"),
    compiler_params=_SC_PARAMS,
    scratch_types=[
        pltpu.VMEM((CHD, KD), jnp.int32),
        pltpu.VMEM((CHD, KD), jnp.float32),
        pltpu.VMEM((KD, LN), jnp.float32),
        pltpu.VMEM((RZ, LN), jnp.float32),
        pltpu.VMEM_SHARED((NP, LN), jnp.float32),
    ],
)


def _sc_spmm_body(hq0, hq1, hq2, hq3, src_hbm, dst_hbm, ew_hbm, out_hbm,
                  src_v, dst_v, ew_v, r0, r1, gsem, ssem, z_v, acc):
  cid = lax.axis_index("c")
  sid = lax.axis_index("s")

  pltpu.sync_copy(src_hbm.at[sid], src_v)
  pltpu.sync_copy(dst_hbm.at[sid], dst_v)
  pltpu.sync_copy(ew_hbm.at[sid], ew_v)
  _zero_rows(z_v, ZR, Q)
  rbufs = (r0, r1)

  def run(hs_ref, qi):
    for r in range(RZ // ZR):
      pltpu.sync_copy(z_v, acc.at[pl.ds(sid * RZ + r * ZR, ZR)])
    plsc.subcore_barrier()

    def g_issue(g, b):
      pltpu.async_copy(hs_ref.at[src_v.at[g]], rbufs[b], gsem)

    def g_wait(g, b):
      pltpu.make_async_copy(hs_ref.at[src_v.at[g]], rbufs[b], gsem).wait()

    def s_issue(g, b):
      pltpu.async_copy(rbufs[b], acc.at[dst_v.at[g]], ssem, add=True)

    def s_wait(g, b):
      pltpu.make_async_copy(rbufs[b], acc.at[dst_v.at[g]], ssem).wait()

    def scale(rows_b, g):
      base = g * KE

      def sbody(it, _, rows_b=rows_b):
        for u in range(4):
          e = it * 4 + u
          w = plsc.load_gather(ew_v, [jnp.full((LN,), base + e, jnp.int32)])
          for f in range(Q // LN):
            rows_b[e, pl.ds(f * LN, LN)] = rows_b[e, pl.ds(f * LN, LN)] * w
        return 0

      lax.fori_loop(0, KE // 4, sbody, 0)

    g_issue(0, 0)

    def pair(hh, _):
      for b in range(2):
        g = hh * 2 + b
        g_wait(g, b)

        @pl.when(g >= 1)
        def _(g=g, b=b):
          s_wait(g - 1, 1 - b)

        @pl.when(g + 1 < CHE)
        def _(g=g, b=b):
          g_issue(g + 1, 1 - b)

        scale(rbufs[b], g)
        s_issue(g, b)
      return 0

    lax.fori_loop(0, CHE // 2, pair, 0)
    s_wait(CHE - 1, 1)
    plsc.subcore_barrier()
    _writeout(acc, out_hbm.at[qi], sid)

  quarters = (hq0, hq1, hq2, hq3)
  for p in range(2):
    for c in range(NC):
      @pl.when(cid == c)
      def _(p=p, c=c):
        run(quarters[2 * p + c], 2 * p + c)


_sc_spmm = pl.kernel(
    _sc_spmm_body,
    out_type=jax.ShapeDtypeStruct((NQ, N, Q), jnp.float32),
    mesh=plsc.VectorSubcoreMesh(core_axis_name="c", subcore_axis_name="s"),
    compiler_params=_SC_PARAMS,
    scratch_types=[
        pltpu.VMEM((CHE, KE), jnp.int32),
        pltpu.VMEM((CHE, KE), jnp.int32),
        pltpu.VMEM((CHE * KE,), jnp.float32),
        pltpu.VMEM((KE, Q), jnp.float32),
        pltpu.VMEM((KE, Q), jnp.float32),
        pltpu.SemaphoreType.DMA,
        pltpu.SemaphoreType.DMA,
        pltpu.VMEM((ZR, Q), jnp.float32),
        pltpu.VMEM_SHARED((NP, Q), jnp.float32),
    ],
)


BR = 2000
NB = N // BR


def _lrelu(h):
  return jnp.where(h >= 0, h, 0.01 * h)


def _row_spec(cols):
  return pl.BlockSpec((BR, cols), lambda i: (i, 0))


def _full_spec(shape):
  nd = len(shape)
  return pl.BlockSpec(shape, lambda i: (0,) * nd)


_QSPECS = [_row_spec(Q) for _ in range(NQ)]


def _split_q(hs, hs_refs):
  for q in range(NQ):
    hs_refs[q][...] = hs[:, q * Q:(q + 1) * Q]


def _tc1_body(degp_ref, x_ref, w1_ref, dis_ref, h0, h1, h2, h3):
  degs = degp_ref[0] + degp_ref[1]
  deg = 1.0 + degs[:, 0:1]
  dis = jnp.where(deg > 0, lax.rsqrt(deg), 0.0)
  dis_ref[...] = dis
  xw = jnp.dot(x_ref[...], w1_ref[...], preferred_element_type=jnp.float32)
  _split_q(xw * dis, (h0, h1, h2, h3))


_tc1 = pl.pallas_call(
    _tc1_body,
    grid=(NB,),
    in_specs=[
        pl.BlockSpec((NC, BR, LN), lambda i: (0, i, 0)),
        _row_spec(DIN),
        _full_spec((DIN, HID)),
    ],
    out_specs=[_row_spec(1)] + _QSPECS,
    out_shape=(
        jax.ShapeDtypeStruct((N, 1), jnp.float32),
    ) + tuple(jax.ShapeDtypeStruct((N, Q), jnp.float32) for _ in range(NQ)),
)


def _tc_bn1_body(s_ref, hq0, hq1, hq2, hq3, dis_ref, b_ref, a_ref, sums_ref):
  i = pl.program_id(0)
  sc = jnp.concatenate([s_ref[q] for q in range(NQ)], axis=1)
  hc = jnp.concatenate([h[...] for h in (hq0, hq1, hq2, hq3)], axis=1)
  pre = dis_ref[...] * (sc + hc) + b_ref[...][None, :]
  a = _lrelu(pre)
  a_ref[...] = a

  @pl.when(i == 0)
  def _():
    sums_ref[...] = jnp.zeros((8, HID), jnp.float32)

  sums_ref[0:1, :] = sums_ref[0:1, :] + jnp.sum(a, axis=0, keepdims=True)
  sums_ref[1:2, :] = sums_ref[1:2, :] + jnp.sum(a * a, axis=0, keepdims=True)


_tc_bn1 = pl.pallas_call(
    _tc_bn1_body,
    grid=(NB,),
    in_specs=[
        pl.BlockSpec((NQ, BR, Q), lambda i: (0, i, 0)),
        *_QSPECS,
        _row_spec(1),
        _full_spec((HID,)),
    ],
    out_specs=[_row_spec(HID), _full_spec((8, HID))],
    out_shape=(
        jax.ShapeDtypeStruct((N, HID), jnp.float32),
        jax.ShapeDtypeStruct((8, HID), jnp.float32),
    ),
)


def _tc_bn2_body(a_ref, sums_ref, dis_ref, g_ref, be_ref, w_ref,
                 out_ref, n0, n1, n2, n3):
  mu = sums_ref[0:1, :] / N
  var = sums_ref[1:2, :] / N - mu * mu
  o = ((a_ref[...] - mu) * lax.rsqrt(var + 1e-5) * g_ref[...][None, :]
       + be_ref[...][None, :])
  out_ref[...] = o
  hsn = jnp.dot(o, w_ref[...], preferred_element_type=jnp.float32)
  _split_q(hsn * dis_ref[...], (n0, n1, n2, n3))


_tc_bn2 = pl.pallas_call(
    _tc_bn2_body,
    grid=(NB,),
    in_specs=[
        _row_spec(HID),
        _full_spec((8, HID)),
        _row_spec(1),
        _full_spec((HID,)),
        _full_spec((HID,)),
        _full_spec((HID, HID)),
    ],
    out_specs=[_row_spec(HID)] + _QSPECS,
    out_shape=(
        jax.ShapeDtypeStruct((N, HID), jnp.float32),
    ) + tuple(jax.ShapeDtypeStruct((N, Q), jnp.float32) for _ in range(NQ)),
)


def _tc_fin_body(s_ref, hq0, hq1, hq2, hq3, dis_ref, b_ref, o1_ref, o2_ref,
                 wl_ref, bl_ref, res_ref):
  sc = jnp.concatenate([s_ref[q] for q in range(NQ)], axis=1)
  hc = jnp.concatenate([h[...] for h in (hq0, hq1, hq2, hq3)], axis=1)
  o3 = _lrelu(dis_ref[...] * (sc + hc) + b_ref[...][None, :])
  wl = wl_ref[...]
  res = (jnp.dot(o1_ref[...], wl[:HID], preferred_element_type=jnp.float32)
         + jnp.dot(o2_ref[...], wl[HID:2 * HID],
                   preferred_element_type=jnp.float32)
         + jnp.dot(o3, wl[2 * HID:], preferred_element_type=jnp.float32)
         + bl_ref[...][None, :])
  res_ref[...] = res


_tc_fin = pl.pallas_call(
    _tc_fin_body,
    grid=(NB,),
    in_specs=[
        pl.BlockSpec((NQ, BR, Q), lambda i: (0, i, 0)),
        *_QSPECS,
        _row_spec(1),
        _full_spec((HID,)),
        _row_spec(HID),
        _row_spec(HID),
        _full_spec((3 * HID, NCLS)),
        _full_spec((NCLS,)),
    ],
    out_specs=_row_spec(NCLS),
    out_shape=jax.ShapeDtypeStruct((N, NCLS), jnp.float32),
)


def kernel(x, edge_index, edge_weights, W1, b1, g1, be1, W2, b2, g2, be2,
           W3, b3, Wl, bl):
  ei = edge_index.astype(jnp.int32)
  src = ei[0]
  dst = ei[1]
  ew = edge_weights.astype(jnp.float32)

  dst_d = dst.reshape(NC * NS, CHD, KD)
  ew_d = ew.reshape(NC * NS, CHD, KD)
  src_s = src.reshape(NS, CHE, KE)
  dst_s = dst.reshape(NS, CHE, KE)
  ew_s = ew.reshape(NS, CHE * KE)

  degp = _sc_deg(dst_d, ew_d)
  dis, *hs1 = _tc1(degp, x, W1)

  s1 = _sc_spmm(*hs1, src_s, dst_s, ew_s)
  a1, sums1 = _tc_bn1(s1, *hs1, dis, b1)
  out1, *hs2 = _tc_bn2(a1, sums1, dis, g1, be1, W2)

  s2 = _sc_spmm(*hs2, src_s, dst_s, ew_s)
  a2, sums2 = _tc_bn1(s2, *hs2, dis, b2)
  out2, *hs3 = _tc_bn2(a2, sums2, dis, g2, be2, W3)

  s3 = _sc_spmm(*hs3, src_s, dst_s, ew_s)
  return _tc_fin(s3, *hs3, dis, b3, out1, out2, Wl, bl)


# speedup vs baseline: 9.9692x; 1.0067x over previous
import jax
import jax.numpy as jnp
from jax import lax
from jax.experimental import pallas as pl
from jax.experimental.pallas import tpu as pltpu
from jax.experimental.pallas import tpu_sc as plsc

N = 10000
E = 320000
DIN = 128
HID = 256
NCLS = 64

NC = 2
NS = 16
LN = 16
NQ = 4
Q = HID // NQ

KD = 80
CHD = E // (NC * NS) // KD
KE = 100
CHE = E // NS // KE
NP = 10240
RZ = NP // NS
ZR = 128

_SC_PARAMS = pltpu.CompilerParams(
    needs_layout_passes=False, use_tc_tiling_on_sc=False)


def _bcast_ld(ref, row_idx, col):
  return plsc.load_gather(ref, [row_idx, jnp.full((LN,), col, jnp.int32)])


def _zero_rows(zbuf, nrows, ncols):
  zv = jnp.zeros((LN,), jnp.float32)

  def body(i, _):
    for f in range(ncols // LN):
      zbuf[i, pl.ds(f * LN, LN)] = zv
    return 0

  lax.fori_loop(0, nrows, body, 0)


RLAST = N - (NS - 1) * RZ


def _writeout(acc, out_ref, sid):
  @pl.when(sid < NS - 1)
  def _():
    pltpu.sync_copy(acc.at[pl.ds(sid * RZ, RZ)], out_ref.at[pl.ds(sid * RZ, RZ)])

  @pl.when(sid == NS - 1)
  def _():
    pltpu.sync_copy(acc.at[pl.ds((NS - 1) * RZ, RLAST)],
                    out_ref.at[pl.ds((NS - 1) * RZ, RLAST)])


def _sc_deg_body(dst_hbm, ew_hbm, out_hbm, dst_v, ew_v, val_v, z_v, acc):
  cid = lax.axis_index("c")
  sid = lax.axis_index("s")
  wid = sid * NC + cid

  _zero_rows(z_v, RZ, LN)
  pltpu.sync_copy(z_v, acc.at[pl.ds(sid * RZ, RZ)])
  pltpu.sync_copy(dst_hbm.at[wid], dst_v)
  pltpu.sync_copy(ew_hbm.at[wid], ew_v)
  plsc.subcore_barrier()

  def chunk(g, _):
    gv = jnp.full((LN,), g, jnp.int32)
    for e in range(KD):
      val_v[e] = _bcast_ld(ew_v, gv, e)
    pltpu.sync_copy(val_v, acc.at[dst_v.at[g]], add=True)
    return 0

  lax.fori_loop(0, CHD, chunk, 0)
  plsc.subcore_barrier()
  _writeout(acc, out_hbm.at[cid], sid)


_sc_deg = pl.kernel(
    _sc_deg_body,
    out_type=jax.ShapeDtypeStruct((NC, N, LN), jnp.float32),
    mesh=plsc.VectorSubcoreMesh(core_axis_name="c", subcore_axis_name="s"),
    compiler_params=_SC_PARAMS,
    scratch_types=[
        pltpu.VMEM((CHD, KD), jnp.int32),
        pltpu.VMEM((CHD, KD), jnp.float32),
        pltpu.VMEM((KD, LN), jnp.float32),
        pltpu.VMEM((RZ, LN), jnp.float32),
        pltpu.VMEM_SHARED((NP, LN), jnp.float32),
    ],
)


def _sc_spmm_body(hq0, hq1, hq2, hq3, src_hbm, dst_hbm, ew_hbm, out_hbm,
                  src_v, dst_v, ew_v, r0, r1, gsem, ssem, z_v, acc):
  cid = lax.axis_index("c")
  sid = lax.axis_index("s")

  pltpu.sync_copy(src_hbm.at[sid], src_v)
  pltpu.sync_copy(dst_hbm.at[sid], dst_v)
  pltpu.sync_copy(ew_hbm.at[sid], ew_v)
  _zero_rows(z_v, ZR, Q)
  rbufs = (r0, r1)

  def run(hs_ref, qi):
    for r in range(RZ // ZR):
      pltpu.sync_copy(z_v, acc.at[pl.ds(sid * RZ + r * ZR, ZR)])
    plsc.subcore_barrier()

    def g_issue(g, b):
      pltpu.async_copy(hs_ref.at[src_v.at[g]], rbufs[b], gsem)

    def g_wait(g, b):
      pltpu.make_async_copy(hs_ref.at[src_v.at[g]], rbufs[b], gsem).wait()

    def s_issue(g, b):
      pltpu.async_copy(rbufs[b], acc.at[dst_v.at[g]], ssem, add=True)

    def s_wait(g, b):
      pltpu.make_async_copy(rbufs[b], acc.at[dst_v.at[g]], ssem).wait()

    def scale(rows_b, g):
      base = g * KE

      def sbody(it, _, rows_b=rows_b):
        for u in range(10):
          e = it * 10 + u
          w = plsc.load_gather(ew_v, [jnp.full((LN,), base + e, jnp.int32)])
          for f in range(Q // LN):
            rows_b[e, pl.ds(f * LN, LN)] = rows_b[e, pl.ds(f * LN, LN)] * w
        return 0

      lax.fori_loop(0, KE // 10, sbody, 0)

    g_issue(0, 0)

    def pair(hh, _):
      for b in range(2):
        g = hh * 2 + b
        g_wait(g, b)

        @pl.when(g >= 1)
        def _(g=g, b=b):
          s_wait(g - 1, 1 - b)

        @pl.when(g + 1 < CHE)
        def _(g=g, b=b):
          g_issue(g + 1, 1 - b)

        scale(rbufs[b], g)
        s_issue(g, b)
      return 0

    lax.fori_loop(0, CHE // 2, pair, 0)
    s_wait(CHE - 1, 1)
    plsc.subcore_barrier()
    _writeout(acc, out_hbm.at[qi], sid)

  quarters = (hq0, hq1, hq2, hq3)
  for p in range(2):
    for c in range(NC):
      @pl.when(cid == c)
      def _(p=p, c=c):
        run(quarters[2 * p + c], 2 * p + c)


_sc_spmm = pl.kernel(
    _sc_spmm_body,
    out_type=jax.ShapeDtypeStruct((NQ, N, Q), jnp.float32),
    mesh=plsc.VectorSubcoreMesh(core_axis_name="c", subcore_axis_name="s"),
    compiler_params=_SC_PARAMS,
    scratch_types=[
        pltpu.VMEM((CHE, KE), jnp.int32),
        pltpu.VMEM((CHE, KE), jnp.int32),
        pltpu.VMEM((CHE * KE,), jnp.float32),
        pltpu.VMEM((KE, Q), jnp.float32),
        pltpu.VMEM((KE, Q), jnp.float32),
        pltpu.SemaphoreType.DMA,
        pltpu.SemaphoreType.DMA,
        pltpu.VMEM((ZR, Q), jnp.float32),
        pltpu.VMEM_SHARED((NP, Q), jnp.float32),
    ],
)


BR = 2000
NB = N // BR


def _lrelu(h):
  return jnp.where(h >= 0, h, 0.01 * h)


def _row_spec(cols):
  return pl.BlockSpec((BR, cols), lambda i: (i, 0))


def _full_spec(shape):
  nd = len(shape)
  return pl.BlockSpec(shape, lambda i: (0,) * nd)


_QSPECS = [_row_spec(Q) for _ in range(NQ)]


def _split_q(hs, hs_refs):
  for q in range(NQ):
    hs_refs[q][...] = hs[:, q * Q:(q + 1) * Q]


def _tc_mm1_body(x_ref, w1_ref, xw_ref):
  xw_ref[...] = jnp.dot(x_ref[...], w1_ref[...],
                        preferred_element_type=jnp.float32)


_tc_mm1 = pl.pallas_call(
    _tc_mm1_body,
    grid=(NB,),
    in_specs=[_row_spec(DIN), _full_spec((DIN, HID))],
    out_specs=_row_spec(HID),
    out_shape=jax.ShapeDtypeStruct((N, HID), jnp.float32),
)


def _tc1_body(degp_ref, xw_ref, dis_ref, h0, h1, h2, h3):
  degs = degp_ref[0] + degp_ref[1]
  deg = 1.0 + degs[:, 0:1]
  dis = jnp.where(deg > 0, lax.rsqrt(deg), 0.0)
  dis_ref[...] = dis
  _split_q(xw_ref[...] * dis, (h0, h1, h2, h3))


_tc1 = pl.pallas_call(
    _tc1_body,
    grid=(NB,),
    in_specs=[
        pl.BlockSpec((NC, BR, LN), lambda i: (0, i, 0)),
        _row_spec(HID),
    ],
    out_specs=[_row_spec(1)] + _QSPECS,
    out_shape=(
        jax.ShapeDtypeStruct((N, 1), jnp.float32),
    ) + tuple(jax.ShapeDtypeStruct((N, Q), jnp.float32) for _ in range(NQ)),
)


def _tc_bn1_body(s_ref, hq0, hq1, hq2, hq3, dis_ref, b_ref, a_ref, sums_ref):
  i = pl.program_id(0)
  sc = jnp.concatenate([s_ref[q] for q in range(NQ)], axis=1)
  hc = jnp.concatenate([h[...] for h in (hq0, hq1, hq2, hq3)], axis=1)
  pre = dis_ref[...] * (sc + hc) + b_ref[...][None, :]
  a = _lrelu(pre)
  a_ref[...] = a

  @pl.when(i == 0)
  def _():
    sums_ref[...] = jnp.zeros((8, HID), jnp.float32)

  sums_ref[0:1, :] = sums_ref[0:1, :] + jnp.sum(a, axis=0, keepdims=True)
  sums_ref[1:2, :] = sums_ref[1:2, :] + jnp.sum(a * a, axis=0, keepdims=True)


_tc_bn1 = pl.pallas_call(
    _tc_bn1_body,
    grid=(NB,),
    in_specs=[
        pl.BlockSpec((NQ, BR, Q), lambda i: (0, i, 0)),
        *_QSPECS,
        _row_spec(1),
        _full_spec((HID,)),
    ],
    out_specs=[_row_spec(HID), _full_spec((8, HID))],
    out_shape=(
        jax.ShapeDtypeStruct((N, HID), jnp.float32),
        jax.ShapeDtypeStruct((8, HID), jnp.float32),
    ),
)


def _tc_bn2_body(a_ref, sums_ref, dis_ref, g_ref, be_ref, w_ref,
                 out_ref, n0, n1, n2, n3):
  mu = sums_ref[0:1, :] / N
  var = sums_ref[1:2, :] / N - mu * mu
  o = ((a_ref[...] - mu) * lax.rsqrt(var + 1e-5) * g_ref[...][None, :]
       + be_ref[...][None, :])
  out_ref[...] = o
  hsn = jnp.dot(o, w_ref[...], preferred_element_type=jnp.float32)
  _split_q(hsn * dis_ref[...], (n0, n1, n2, n3))


_tc_bn2 = pl.pallas_call(
    _tc_bn2_body,
    grid=(NB,),
    in_specs=[
        _row_spec(HID),
        _full_spec((8, HID)),
        _row_spec(1),
        _full_spec((HID,)),
        _full_spec((HID,)),
        _full_spec((HID, HID)),
    ],
    out_specs=[_row_spec(HID)] + _QSPECS,
    out_shape=(
        jax.ShapeDtypeStruct((N, HID), jnp.float32),
    ) + tuple(jax.ShapeDtypeStruct((N, Q), jnp.float32) for _ in range(NQ)),
)


def _tc_fin_body(s_ref, hq0, hq1, hq2, hq3, dis_ref, b_ref, o1_ref, o2_ref,
                 wl_ref, bl_ref, res_ref):
  sc = jnp.concatenate([s_ref[q] for q in range(NQ)], axis=1)
  hc = jnp.concatenate([h[...] for h in (hq0, hq1, hq2, hq3)], axis=1)
  o3 = _lrelu(dis_ref[...] * (sc + hc) + b_ref[...][None, :])
  wl = wl_ref[...]
  res = (jnp.dot(o1_ref[...], wl[:HID], preferred_element_type=jnp.float32)
         + jnp.dot(o2_ref[...], wl[HID:2 * HID],
                   preferred_element_type=jnp.float32)
         + jnp.dot(o3, wl[2 * HID:], preferred_element_type=jnp.float32)
         + bl_ref[...][None, :])
  res_ref[...] = res


_tc_fin = pl.pallas_call(
    _tc_fin_body,
    grid=(NB,),
    in_specs=[
        pl.BlockSpec((NQ, BR, Q), lambda i: (0, i, 0)),
        *_QSPECS,
        _row_spec(1),
        _full_spec((HID,)),
        _row_spec(HID),
        _row_spec(HID),
        _full_spec((3 * HID, NCLS)),
        _full_spec((NCLS,)),
    ],
    out_specs=_row_spec(NCLS),
    out_shape=jax.ShapeDtypeStruct((N, NCLS), jnp.float32),
)


def kernel(x, edge_index, edge_weights, W1, b1, g1, be1, W2, b2, g2, be2,
           W3, b3, Wl, bl):
  ei = edge_index.astype(jnp.int32)
  src = ei[0]
  dst = ei[1]
  ew = edge_weights.astype(jnp.float32)

  dst_d = dst.reshape(NC * NS, CHD, KD)
  ew_d = ew.reshape(NC * NS, CHD, KD)
  src_s = src.reshape(NS, CHE, KE)
  dst_s = dst.reshape(NS, CHE, KE)
  ew_s = ew.reshape(NS, CHE * KE)

  xw1 = _tc_mm1(x, W1)
  degp = _sc_deg(dst_d, ew_d)
  dis, *hs1 = _tc1(degp, xw1)

  s1 = _sc_spmm(*hs1, src_s, dst_s, ew_s)
  a1, sums1 = _tc_bn1(s1, *hs1, dis, b1)
  out1, *hs2 = _tc_bn2(a1, sums1, dis, g1, be1, W2)

  s2 = _sc_spmm(*hs2, src_s, dst_s, ew_s)
  a2, sums2 = _tc_bn1(s2, *hs2, dis, b2)
  out2, *hs3 = _tc_bn2(a2, sums2, dis, g2, be2, W3)

  s3 = _sc_spmm(*hs3, src_s, dst_s, ew_s)
  return _tc_fin(s3, *hs3, dis, b3, out1, out2, Wl, bl)
